# Initial kernel scaffold; baseline (speedup 1.0000x reference)
import jax
import jax.numpy as jnp
from jax.experimental import pallas as pl


def kernel(input_ids, token_table, pos_table, ln_gamma, ln_beta):
    raise NotImplementedError("write your pallas kernel here")


# speedup vs baseline: 1.4534x; 1.4534x over previous
import functools

import jax
import jax.numpy as jnp
from jax import lax
from jax.experimental import pallas as pl
from jax.experimental.pallas import tpu as pltpu
from jax.experimental.pallas import tpu_sc as plsc

_VOCAB = 100000
_D = 1024
_B = 4
_L = 2048
_N = _B * _L

_NC = 2
_NS = 16
_NW = _NC * _NS
_RPW = _N // _NW
_CH = 32
_NCH = _RPW // _CH

_ROWS = 256
_GRID = _N // _ROWS


def _gather_body(ids_hbm, table_hbm, out_hbm, idx_v, buf0, buf1,
                 gsem0, gsem1, osem0, osem1):
    wid = lax.axis_index("s") * _NC + lax.axis_index("c")
    base = wid * _RPW
    pltpu.sync_copy(ids_hbm.at[wid], idx_v)
    bufs = (buf0, buf1)
    gsems = (gsem0, gsem1)
    osems = (osem0, osem1)
    ghandles = [None, None]
    ohandles = [None, None]
    ghandles[0] = pltpu.async_copy(table_hbm.at[idx_v.at[0]], bufs[0], gsems[0])
    for c in range(_NCH):
        s = c % 2
        if c + 1 < _NCH:
            s2 = (c + 1) % 2
            if ohandles[s2] is not None:
                ohandles[s2].wait()
                ohandles[s2] = None
            ghandles[s2] = pltpu.async_copy(
                table_hbm.at[idx_v.at[c + 1]], bufs[s2], gsems[s2])
        ghandles[s].wait()
        ohandles[s] = pltpu.async_copy(
            bufs[s], out_hbm.at[pl.ds(base + c * _CH, _CH)], osems[s])
    for h in ohandles:
        if h is not None:
            h.wait()


@jax.jit
def _gather(ids3, table):
    mesh = plsc.VectorSubcoreMesh(core_axis_name="c", subcore_axis_name="s")
    return pl.kernel(
        _gather_body,
        mesh=mesh,
        out_type=jax.ShapeDtypeStruct((_N, _D), jnp.float32),
        scratch_types=[
            pltpu.VMEM((_NCH, _CH), jnp.int32),
            pltpu.VMEM((_CH, _D), jnp.float32),
            pltpu.VMEM((_CH, _D), jnp.float32),
            pltpu.SemaphoreType.DMA,
            pltpu.SemaphoreType.DMA,
            pltpu.SemaphoreType.DMA,
            pltpu.SemaphoreType.DMA,
        ],
    )(ids3, table)


def _dense_body(e_ref, pos_ref, gam_ref, beta_ref, xs_ref, t_ref):
    e = e_ref[...] + pos_ref[...]
    mu = jnp.mean(e, axis=1, keepdims=True)
    d = e - mu
    var = jnp.mean(d * d, axis=1, keepdims=True)
    y = d * lax.rsqrt(var + 1e-5) * gam_ref[...] + beta_ref[...]
    n2 = jnp.sum(y * y, axis=1, keepdims=True)
    nrm = jnp.sqrt(n2)
    scale = jnp.where(nrm > 2.0, 2.0 / jnp.maximum(nrm, 1e-8), 1.0)
    e2 = y * scale
    e2 = jnp.clip(jnp.where(jnp.isnan(e2), 0.0, e2), -10000.0, 10000.0)
    vn2 = jnp.sum(e2 * e2, axis=1, keepdims=True)
    vn = jnp.maximum(jnp.sqrt(vn2), 1e-8)
    ex = jnp.exp(vn)
    sinh_vn = 0.5 * (ex - 1.0 / ex)
    sfac = sinh_vn / vn
    xs = sfac * e2
    s2 = jnp.sum(xs * xs, axis=1, keepdims=True)
    t = jnp.sqrt(1.0 + s2)
    xs = jnp.clip(jnp.where(jnp.isnan(xs), 0.0, xs), -10000.0, 10000.0)
    t = jnp.clip(jnp.where(jnp.isnan(t), 0.0, t), -10000.0, 10000.0)
    xs_ref[...] = xs
    t_ref[...] = t


_dense_call = pl.pallas_call(
    _dense_body,
    grid=(_GRID,),
    in_specs=[
        pl.BlockSpec((_ROWS, _D), lambda i: (i, 0)),
        pl.BlockSpec((_ROWS, _D), lambda i: (i % (_L // _ROWS), 0)),
        pl.BlockSpec((1, _D), lambda i: (0, 0)),
        pl.BlockSpec((1, _D), lambda i: (0, 0)),
    ],
    out_specs=[
        pl.BlockSpec((_ROWS, _D), lambda i: (i, 0)),
        pl.BlockSpec((_ROWS, 1), lambda i: (i, 0)),
    ],
    out_shape=[
        jax.ShapeDtypeStruct((_N, _D), jnp.float32),
        jax.ShapeDtypeStruct((_N, 1), jnp.float32),
    ],
)


def kernel(input_ids, token_table, pos_table, ln_gamma, ln_beta):
    Bp, Lp = input_ids.shape
    ids3 = input_ids.astype(jnp.int32).reshape(_NW, _NCH, _CH)
    gathered = _gather(ids3, token_table)
    xs, t = _dense_call(gathered, pos_table[:Lp],
                        ln_gamma.reshape(1, _D), ln_beta.reshape(1, _D))
    x = jnp.concatenate([t, xs], axis=1)
    return x.reshape(Bp, Lp, _D + 1)


# speedup vs baseline: 1.7895x; 1.2312x over previous
import functools

import jax
import jax.numpy as jnp
from jax import lax
from jax.experimental import pallas as pl
from jax.experimental.pallas import tpu as pltpu
from jax.experimental.pallas import tpu_sc as plsc

_VOCAB = 100000
_D = 1024
_B = 4
_L = 2048
_N = _B * _L

_NC = 2
_NS = 16
_NW = _NC * _NS
_RPW = _N // _NW
_CH = 32
_NCH = _RPW // _CH

_ROWS = 256
_GRID = _N // _ROWS


def _gather_body(ids_hbm, table_hbm, out_hbm, idx_v, buf0, buf1,
                 gsem0, gsem1, osem0, osem1):
    wid = lax.axis_index("s") * _NC + lax.axis_index("c")
    base = wid * _RPW
    pltpu.sync_copy(ids_hbm.at[wid], idx_v)
    bufs = (buf0, buf1)
    gsems = (gsem0, gsem1)
    osems = (osem0, osem1)
    ghandles = [None, None]
    ohandles = [None, None]
    ghandles[0] = pltpu.async_copy(table_hbm.at[idx_v.at[0]], bufs[0], gsems[0])
    for c in range(_NCH):
        s = c % 2
        if c + 1 < _NCH:
            s2 = (c + 1) % 2
            if ohandles[s2] is not None:
                ohandles[s2].wait()
                ohandles[s2] = None
            ghandles[s2] = pltpu.async_copy(
                table_hbm.at[idx_v.at[c + 1]], bufs[s2], gsems[s2])
        ghandles[s].wait()
        ohandles[s] = pltpu.async_copy(
            bufs[s], out_hbm.at[pl.ds(base + c * _CH, _CH)], osems[s])
    for h in ohandles:
        if h is not None:
            h.wait()


@jax.jit
def _gather(ids3, table):
    mesh = plsc.VectorSubcoreMesh(core_axis_name="c", subcore_axis_name="s")
    return pl.kernel(
        _gather_body,
        mesh=mesh,
        out_type=jax.ShapeDtypeStruct((_N, _D), jnp.float32),
        scratch_types=[
            pltpu.VMEM((_NCH, _CH), jnp.int32),
            pltpu.VMEM((_CH, _D), jnp.float32),
            pltpu.VMEM((_CH, _D), jnp.float32),
            pltpu.SemaphoreType.DMA,
            pltpu.SemaphoreType.DMA,
            pltpu.SemaphoreType.DMA,
            pltpu.SemaphoreType.DMA,
        ],
    )(ids3, table)


def _dense_body(e_ref, pos_ref, gam_ref, beta_ref, out_ref):
    e = e_ref[...] + pos_ref[...]
    mu = jnp.mean(e, axis=1, keepdims=True)
    d = e - mu
    var = jnp.mean(d * d, axis=1, keepdims=True)
    y = d * lax.rsqrt(var + 1e-5) * gam_ref[...] + beta_ref[...]
    n2 = jnp.sum(y * y, axis=1, keepdims=True)
    nrm = jnp.sqrt(n2)
    scale = jnp.where(nrm > 2.0, 2.0 / jnp.maximum(nrm, 1e-8), 1.0)
    e2 = y * scale
    e2 = jnp.clip(jnp.where(jnp.isnan(e2), 0.0, e2), -10000.0, 10000.0)
    vn2 = jnp.sum(e2 * e2, axis=1, keepdims=True)
    vn = jnp.maximum(jnp.sqrt(vn2), 1e-8)
    ex = jnp.exp(vn)
    sinh_vn = 0.5 * (ex - 1.0 / ex)
    sfac = sinh_vn / vn
    xs = sfac * e2
    s2 = jnp.sum(xs * xs, axis=1, keepdims=True)
    t = jnp.sqrt(1.0 + s2)
    xs = jnp.clip(jnp.where(jnp.isnan(xs), 0.0, xs), -10000.0, 10000.0)
    t = jnp.clip(jnp.where(jnp.isnan(t), 0.0, t), -10000.0, 10000.0)
    out_ref[...] = jnp.concatenate([t, xs], axis=1)


_PB = _L // _ROWS
_dense_call = pl.pallas_call(
    _dense_body,
    grid=(_PB, _B),
    in_specs=[
        pl.BlockSpec((_ROWS, _D), lambda i, j: (j * _PB + i, 0)),
        pl.BlockSpec((_ROWS, _D), lambda i, j: (i, 0)),
        pl.BlockSpec((1, _D), lambda i, j: (0, 0)),
        pl.BlockSpec((1, _D), lambda i, j: (0, 0)),
    ],
    out_specs=pl.BlockSpec((_ROWS, _D + 1), lambda i, j: (j * _PB + i, 0)),
    out_shape=jax.ShapeDtypeStruct((_N, _D + 1), jnp.float32),
)


def kernel(input_ids, token_table, pos_table, ln_gamma, ln_beta):
    Bp, Lp = input_ids.shape
    ids3 = input_ids.astype(jnp.int32).reshape(_NW, _NCH, _CH)
    gathered = _gather(ids3, token_table)
    x = _dense_call(gathered, pos_table[:Lp],
                    ln_gamma.reshape(1, _D), ln_beta.reshape(1, _D))
    return x.reshape(Bp, Lp, _D + 1)


# speedup vs baseline: 1.7941x; 1.0026x over previous
import functools

import jax
import jax.numpy as jnp
from jax import lax
from jax.experimental import pallas as pl
from jax.experimental.pallas import tpu as pltpu
from jax.experimental.pallas import tpu_sc as plsc

_VOCAB = 100000
_D = 1024
_B = 4
_L = 2048
_N = _B * _L

_NC = 2
_NS = 16
_NW = _NC * _NS
_RPW = _N // _NW
_CH = 32
_NCH = _RPW // _CH

_ROWS = 256
_GRID = _N // _ROWS


def _gather_body(ids_hbm, table_hbm, out_hbm, idx_v, buf0, buf1,
                 gsem0, gsem1, osem0, osem1):
    wid = lax.axis_index("s") * _NC + lax.axis_index("c")
    base = wid * _RPW
    pltpu.sync_copy(ids_hbm.at[wid], idx_v)
    bufs = (buf0, buf1)
    gsems = (gsem0, gsem1)
    osems = (osem0, osem1)
    ghandles = [None, None]
    ohandles = [None, None]
    ghandles[0] = pltpu.async_copy(table_hbm.at[idx_v.at[0]], bufs[0], gsems[0])
    for c in range(_NCH):
        s = c % 2
        if c + 1 < _NCH:
            s2 = (c + 1) % 2
            if ohandles[s2] is not None:
                ohandles[s2].wait()
                ohandles[s2] = None
            ghandles[s2] = pltpu.async_copy(
                table_hbm.at[idx_v.at[c + 1]], bufs[s2], gsems[s2])
        ghandles[s].wait()
        ohandles[s] = pltpu.async_copy(
            bufs[s], out_hbm.at[pl.ds(base + c * _CH, _CH)], osems[s])
    for h in ohandles:
        if h is not None:
            h.wait()


@jax.jit
def _gather(ids3, table):
    mesh = plsc.VectorSubcoreMesh(core_axis_name="c", subcore_axis_name="s")
    return pl.kernel(
        _gather_body,
        mesh=mesh,
        compiler_params=pltpu.CompilerParams(use_tc_tiling_on_sc=True),
        out_type=jax.ShapeDtypeStruct((_N, _D), jnp.float32),
        scratch_types=[
            pltpu.VMEM((_NCH, _CH), jnp.int32),
            pltpu.VMEM((_CH, _D), jnp.float32),
            pltpu.VMEM((_CH, _D), jnp.float32),
            pltpu.SemaphoreType.DMA,
            pltpu.SemaphoreType.DMA,
            pltpu.SemaphoreType.DMA,
            pltpu.SemaphoreType.DMA,
        ],
    )(ids3, table)


def _dense_body(e_ref, pos_ref, gam_ref, beta_ref, out_ref):
    e = e_ref[...] + pos_ref[...]
    mu = jnp.mean(e, axis=1, keepdims=True)
    d = e - mu
    var = jnp.mean(d * d, axis=1, keepdims=True)
    y = d * lax.rsqrt(var + 1e-5) * gam_ref[...] + beta_ref[...]
    n2 = jnp.sum(y * y, axis=1, keepdims=True)
    nrm = jnp.sqrt(n2)
    scale = jnp.where(nrm > 2.0, 2.0 / jnp.maximum(nrm, 1e-8), 1.0)
    e2 = y * scale
    e2 = jnp.clip(jnp.where(jnp.isnan(e2), 0.0, e2), -10000.0, 10000.0)
    vn2 = jnp.sum(e2 * e2, axis=1, keepdims=True)
    vn = jnp.maximum(jnp.sqrt(vn2), 1e-8)
    ex = jnp.exp(vn)
    sinh_vn = 0.5 * (ex - 1.0 / ex)
    sfac = sinh_vn / vn
    xs = sfac * e2
    s2 = jnp.sum(xs * xs, axis=1, keepdims=True)
    t = jnp.sqrt(1.0 + s2)
    xs = jnp.clip(jnp.where(jnp.isnan(xs), 0.0, xs), -10000.0, 10000.0)
    t = jnp.clip(jnp.where(jnp.isnan(t), 0.0, t), -10000.0, 10000.0)
    out_ref[...] = jnp.concatenate([t, xs], axis=1)


_PB = _L // _ROWS
_dense_call = pl.pallas_call(
    _dense_body,
    grid=(_PB, _B),
    in_specs=[
        pl.BlockSpec((_ROWS, _D), lambda i, j: (j * _PB + i, 0)),
        pl.BlockSpec((_ROWS, _D), lambda i, j: (i, 0)),
        pl.BlockSpec((1, _D), lambda i, j: (0, 0)),
        pl.BlockSpec((1, _D), lambda i, j: (0, 0)),
    ],
    out_specs=pl.BlockSpec((_ROWS, _D + 1), lambda i, j: (j * _PB + i, 0)),
    out_shape=jax.ShapeDtypeStruct((_N, _D + 1), jnp.float32),
)


def kernel(input_ids, token_table, pos_table, ln_gamma, ln_beta):
    Bp, Lp = input_ids.shape
    ids3 = input_ids.astype(jnp.int32).reshape(_NW, _NCH, _CH)
    gathered = _gather(ids3, token_table)
    x = _dense_call(gathered, pos_table[:Lp],
                    ln_gamma.reshape(1, _D), ln_beta.reshape(1, _D))
    return x.reshape(Bp, Lp, _D + 1)


# speedup vs baseline: 1.8517x; 1.0321x over previous
import functools

import jax
import jax.numpy as jnp
from jax import lax
from jax.experimental import pallas as pl
from jax.experimental.pallas import tpu as pltpu
from jax.experimental.pallas import tpu_sc as plsc

_VOCAB = 100000
_D = 1024
_B = 4
_L = 2048
_N = _B * _L

_NC = 2
_NS = 16
_NW = _NC * _NS
_RPW = _N // _NW
_CH = 32
_NCH = _RPW // _CH

_ROWS = 256
_GRID = _N // _ROWS


def _gather_body(ids_hbm, table_hbm, out_hbm, idx_v, buf0, buf1,
                 gsem0, gsem1, osem0, osem1):
    wid = lax.axis_index("s") * _NC + lax.axis_index("c")
    base = wid * _RPW
    pltpu.sync_copy(ids_hbm.at[wid], idx_v)
    bufs = (buf0, buf1)
    gsems = (gsem0, gsem1)
    osems = (osem0, osem1)
    ghandles = [None, None]
    ohandles = [None, None]
    ghandles[0] = pltpu.async_copy(table_hbm.at[idx_v.at[0]], bufs[0], gsems[0])
    for c in range(_NCH):
        s = c % 2
        if c + 1 < _NCH:
            s2 = (c + 1) % 2
            if ohandles[s2] is not None:
                ohandles[s2].wait()
                ohandles[s2] = None
            ghandles[s2] = pltpu.async_copy(
                table_hbm.at[idx_v.at[c + 1]], bufs[s2], gsems[s2])
        ghandles[s].wait()
        ohandles[s] = pltpu.async_copy(
            bufs[s], out_hbm.at[pl.ds(base + c * _CH, _CH)], osems[s])
    for h in ohandles:
        if h is not None:
            h.wait()


@jax.jit
def _gather(ids3, table):
    mesh = plsc.VectorSubcoreMesh(core_axis_name="c", subcore_axis_name="s")
    return pl.kernel(
        _gather_body,
        mesh=mesh,
        compiler_params=pltpu.CompilerParams(use_tc_tiling_on_sc=True),
        out_type=jax.ShapeDtypeStruct((_N, _D), jnp.float32),
        scratch_types=[
            pltpu.VMEM((_NCH, _CH), jnp.int32),
            pltpu.VMEM((_CH, _D), jnp.float32),
            pltpu.VMEM((_CH, _D), jnp.float32),
            pltpu.SemaphoreType.DMA,
            pltpu.SemaphoreType.DMA,
            pltpu.SemaphoreType.DMA,
            pltpu.SemaphoreType.DMA,
        ],
    )(ids3, table)


def _dense_body(e_ref, pos_ref, gam_ref, beta_ref, out_ref):
    e = e_ref[...] + pos_ref[...]
    s1 = jnp.sum(e, axis=1, keepdims=True)
    sq = jnp.sum(e * e, axis=1, keepdims=True)
    mu = s1 * (1.0 / _D)
    var = jnp.maximum(sq * (1.0 / _D) - mu * mu, 0.0)
    y = (e - mu) * lax.rsqrt(var + 1e-5) * gam_ref[...] + beta_ref[...]
    n2 = jnp.sum(y * y, axis=1, keepdims=True)
    nrm = jnp.sqrt(n2)
    scale = jnp.where(nrm > 2.0, 2.0 / jnp.maximum(nrm, 1e-8), 1.0)
    vn2 = n2 * (scale * scale)
    vn = jnp.maximum(jnp.sqrt(vn2), 1e-8)
    ex = jnp.exp(vn)
    sfac = (0.5 * (ex - 1.0 / ex)) / vn
    xs = y * (sfac * scale)
    t = jnp.sqrt(1.0 + vn2 * (sfac * sfac))
    out_ref[...] = jnp.concatenate([t, xs], axis=1)


_PB = _L // _ROWS
_dense_call = pl.pallas_call(
    _dense_body,
    grid=(_PB, _B),
    in_specs=[
        pl.BlockSpec((_ROWS, _D), lambda i, j: (j * _PB + i, 0)),
        pl.BlockSpec((_ROWS, _D), lambda i, j: (i, 0)),
        pl.BlockSpec((1, _D), lambda i, j: (0, 0)),
        pl.BlockSpec((1, _D), lambda i, j: (0, 0)),
    ],
    out_specs=pl.BlockSpec((_ROWS, _D + 1), lambda i, j: (j * _PB + i, 0)),
    out_shape=jax.ShapeDtypeStruct((_N, _D + 1), jnp.float32),
)


def kernel(input_ids, token_table, pos_table, ln_gamma, ln_beta):
    Bp, Lp = input_ids.shape
    ids3 = input_ids.astype(jnp.int32).reshape(_NW, _NCH, _CH)
    gathered = _gather(ids3, token_table)
    x = _dense_call(gathered, pos_table[:Lp],
                    ln_gamma.reshape(1, _D), ln_beta.reshape(1, _D))
    return x.reshape(Bp, Lp, _D + 1)
